# baseline (speedup 1.0000x reference)
import functools

import jax
import jax.numpy as jnp
from jax import lax
from jax.experimental import pallas as pl
from jax.experimental.pallas import tpu as pltpu
from jax.experimental.pallas import tpu_sc as plsc

N = 10000
E = 320000
G = 64
NC = 2
NS = 16
NW = NC * NS
CH = 80
CAP = E + 80
NCHB = CAP // CH
CPB = NCHB * 3 * CH
EPW = E // NW
HALF = N // 2


def _iota16():
    return lax.broadcasted_iota(jnp.int32, (16,), 0)


def _bucket_of(d16):
    return lax.shift_right_logical((d16 + d16) * 6711, 22)


def _div80(c):
    return lax.shift_right_logical(lax.shift_right_logical(c, 4) * 26215, 17)


def _wid():
    return lax.axis_index("c") * NS + lax.axis_index("s")


def _count_body(dst_hbm, cm_out, dv, stg):
    w = _wid()
    iota = _iota16()

    def chunk(ci, carry):
        lo, hi = carry
        pltpu.sync_copy(dst_hbm.at[pl.ds(w * EPW + ci * 2000, 2000)], dv)

        def vstep(k, c2):
            lo2, hi2 = c2
            b16 = _bucket_of(dv[pl.ds(k * 16, 16)])
            for i in range(16):
                oh = (iota == i).astype(jnp.int32)
                lo2 = lo2 + plsc.all_reduce_population_count(b16 == i) * oh
                hi2 = hi2 + plsc.all_reduce_population_count(b16 == (16 + i)) * oh
            return lo2, hi2

        return lax.fori_loop(0, 125, vstep, (lo, hi))

    z = jnp.zeros((16,), jnp.int32)
    lo, hi = lax.fori_loop(0, EPW // 2000, chunk, (z, z))
    stg[pl.ds(0, 16)] = lo
    stg[pl.ds(16, 16)] = hi
    pltpu.sync_copy(stg, cm_out.at[pl.ds(w * 32, 32)])


def _div80v(t16):
    return lax.shift_right_logical(lax.shift_right_logical(t16, 4) * 26215, 17)


def _part_body(src_hbm, dst_hbm, ew_hbm, cm_hbm, bpk,
               cmv, sv, dv, wv, slot_v, offs):
    w = _wid()
    iota = _iota16()
    ones = jnp.ones((16,), jnp.int32)
    pltpu.sync_copy(cm_hbm, cmv)
    o_lo = iota * 0
    o_hi = iota * 0

    def accw(wp, carry):
        lo, hi = carry
        m = jnp.where(wp < w, 1, 0)
        lo = lo + cmv[pl.ds(wp * 32, 16)] * m
        hi = hi + cmv[pl.ds(wp * 32 + 16, 16)] * m
        return lo, hi

    o_lo, o_hi = lax.fori_loop(0, NW, accw, (o_lo, o_hi))
    offs[pl.ds(0, 16)] = o_lo
    offs[pl.ds(16, 16)] = o_hi

    BC = 2000

    def chunk(ci, carry):
        e0 = w * EPW + ci * BC
        pltpu.sync_copy(src_hbm.at[pl.ds(e0, BC)], sv)
        pltpu.sync_copy(dst_hbm.at[pl.ds(e0, BC)], dv)
        pltpu.sync_copy(ew_hbm.at[pl.ds(e0, BC)], wv)

        def sub(k2, carry2):
            for k in range(CH // 16):
                sl16 = pl.ds(k2 * CH + k * 16, 16)
                b16 = _bucket_of(dv[sl16])
                rank = jnp.zeros((16,), jnp.int32)
                for m in range(1, 16):
                    sh = b16.at[jnp.maximum(iota - m, 0)].get(mode="promise_in_bounds")
                    rank = rank + jnp.where((iota >= m) & (sh == b16), 1, 0)
                tb = plsc.load_gather(offs, [b16])
                t = tb + rank
                plsc.addupdate_scatter(offs, [b16], ones)
                c0 = _div80v(t)
                addr = b16 * CPB + c0 * 240 + (t - c0 * 80)
                slot_v[pl.ds(k * 16, 16)] = addr
            pltpu.sync_copy(sv.at[pl.ds(k2 * CH, CH)], bpk.at[slot_v])
            for k in range(CH // 16):
                sl16 = pl.ds(k * 16, 16)
                slot_v[sl16] = slot_v[sl16] + 80
            pltpu.sync_copy(dv.at[pl.ds(k2 * CH, CH)], bpk.at[slot_v])
            for k in range(CH // 16):
                sl16 = pl.ds(k * 16, 16)
                slot_v[sl16] = slot_v[sl16] + 80
            pltpu.sync_copy(wv.at[pl.ds(k2 * CH, CH)], bpk.at[slot_v])
            return carry2

        return lax.fori_loop(0, BC // CH, sub, carry)

    lax.fori_loop(0, EPW // BC, chunk, 0)


def _agg_body(bpk, x_hbm, cm_hbm, zeros_hbm, out_hbm,
              cmv, meta2, idx2, dl_v, ew_v, grows2, srows, acc, sem,
              *, mode, passes):
    c = lax.axis_index("c")
    s = lax.axis_index("s")
    iota = _iota16()
    pltpu.sync_copy(cm_hbm, cmv.at[pl.ds(0, NW * 32)])

    if mode == "l1":
        zrows, zextra, zoff = 312, 16, 4992
        worows, woextra, wooff = 312, 8, 4992
        trash = HALF
    else:
        zrows, zextra, zoff = 624, 24, 9984
        worows, woextra, wooff = 624, 16, 9984
        trash = N

    for p in range(passes):
        if mode == "l1":
            buckets = [c * 16 + s]
            rowbase = c * HALF
            go = c * 0
            outbase = c * HALF
        else:
            buckets = [2 * s, 2 * s + 1]
            rowbase = c * 0
            sl = NC * p + c
            go = sl * N
            outbase = sl * N

        pltpu.sync_copy(zeros_hbm.at[pl.ds(0, zrows), :],
                        acc.at[pl.ds(s * zrows, zrows), :])

        @pl.when(s == NS - 1)
        def _():
            pltpu.sync_copy(zeros_hbm.at[pl.ds(0, zextra), :],
                            acc.at[pl.ds(zoff, zextra), :])

        plsc.subcore_barrier()

        go_vec = lax.broadcast_in_dim(go, (16,), ())
        rb_vec = lax.broadcast_in_dim(rowbase, (16,), ())

        for b in buckets:
            cnt = jnp.int32(0)
            for wp in range(NW):
                cnt = cnt + cmv[pl.ds(wp * 32 + b, 16)][0]
            nfull = _div80(cnt)
            ntot = nfull + 1
            r = cnt - nfull * 80
            mbase = b * CPB

            def fetch(ci, p):
                mo = p * 240
                pltpu.sync_copy(bpk.at[pl.ds(mbase + ci * 240, 240)],
                                meta2.at[pl.ds(mo, 240)])
                reff = lax.broadcast_in_dim(
                    jnp.where(ci == nfull, r, CH), (16,), ())
                for k in range(CH // 16):
                    m = (iota + k * 16) < reff
                    sv = meta2[pl.ds(mo + k * 16, 16)]
                    idx2[pl.ds(p * CH + k * 16, 16)] = jnp.where(m, sv + go_vec, 0)
                pltpu.async_copy(x_hbm.at[idx2.at[pl.ds(p * CH, CH)]],
                                 grows2.at[pl.ds(p * CH, CH), :], sem)

            fetch(jnp.int32(0), jnp.int32(0))

            def chunk(ci, carry):
                p = ci % 2
                nx = ci + 1

                @pl.when(nx < ntot)
                def _():
                    fetch(nx, 1 - p)

                mo = p * 240
                reff = lax.broadcast_in_dim(
                    jnp.where(ci == nfull, r, CH), (16,), ())
                for k in range(CH // 16):
                    m = (iota + k * 16) < reff
                    dv = meta2[pl.ds(mo + CH + k * 16, 16)]
                    dl_v[pl.ds(k * 16, 16)] = jnp.where(m, dv - rb_vec, trash)
                    wbits = meta2[pl.ds(mo + 2 * CH + k * 16, 16)]
                    ew_v[pl.ds(k * 16, 16)] = jnp.where(
                        m, plsc.bitcast(wbits, jnp.float32), 0.0)
                pltpu.make_async_copy(x_hbm.at[idx2.at[pl.ds(p * CH, CH)]],
                                      grows2.at[pl.ds(p * CH, CH), :], sem).wait()

                def egroup(q, carry2):
                    evs = ew_v[pl.ds(q * 16, 16)]
                    for rr in range(16):
                        j = q * 16 + rr
                        ev = evs[rr]
                        for f in range(8):
                            fl = pl.ds(f * 16, 16)
                            srows[j, fl] = grows2[p * CH + j, fl] * ev
                    return carry2

                lax.fori_loop(0, CH // 16, egroup, 0)
                pltpu.sync_copy(srows, acc.at[dl_v], add=True)
                return carry

            lax.fori_loop(0, ntot, chunk, 0)

        plsc.subcore_barrier()

        pltpu.sync_copy(acc.at[pl.ds(s * worows, worows), :],
                        out_hbm.at[pl.ds(outbase + s * worows, worows), :])

        @pl.when(s == NS - 1)
        def _():
            pltpu.sync_copy(acc.at[pl.ds(wooff, woextra), :],
                            out_hbm.at[pl.ds(outbase + wooff, woextra), :])
        if p + 1 < passes:
            plsc.subcore_barrier()


_MESH = dict(core_axis_name="c", subcore_axis_name="s",
             num_cores=NC, num_subcores=NS)


def _sc_partition(src, dst, ew):
    cm = pl.kernel(
        _count_body,
        out_type=jax.ShapeDtypeStruct((NW * 32,), jnp.int32),
        mesh=plsc.VectorSubcoreMesh(**_MESH),
        scratch_types=[
            pltpu.VMEM((2000,), jnp.int32),
            pltpu.VMEM((32,), jnp.int32),
        ],
        compiler_params=pltpu.CompilerParams(needs_layout_passes=False),
    )(dst)
    bpk = pl.kernel(
        _part_body,
        out_type=jax.ShapeDtypeStruct((NW * CPB,), jnp.int32),
        mesh=plsc.VectorSubcoreMesh(**_MESH),
        scratch_types=[
            pltpu.VMEM((NW * 32,), jnp.int32),
            pltpu.VMEM((2000,), jnp.int32),
            pltpu.VMEM((2000,), jnp.int32),
            pltpu.VMEM((2000,), jnp.int32),
            pltpu.VMEM((CH,), jnp.int32),
            pltpu.VMEM((32,), jnp.int32),
        ],
        compiler_params=pltpu.CompilerParams(needs_layout_passes=False),
    )(src, dst, ew, cm)
    return bpk, cm


def _sc_agg(xflat, bpk, cm, zeros, *, mode, passes, out_slabs):
    accr = (HALF + 8) if mode == "l1" else (N + 8)
    body = functools.partial(_agg_body, mode=mode, passes=passes)
    return pl.kernel(
        body,
        out_type=jax.ShapeDtypeStruct((out_slabs * N, 128), jnp.float32),
        mesh=plsc.VectorSubcoreMesh(**_MESH),
        scratch_types=[
            pltpu.VMEM((NW * 32 + 16,), jnp.int32),
            pltpu.VMEM((2 * 3 * CH,), jnp.int32),
            pltpu.VMEM((2 * CH,), jnp.int32),
            pltpu.VMEM((CH,), jnp.int32),
            pltpu.VMEM((CH,), jnp.float32),
            pltpu.VMEM((2 * CH, 128), jnp.float32),
            pltpu.VMEM((CH, 128), jnp.float32),
            pltpu.VMEM_SHARED((accr, 128), jnp.float32),
            pltpu.SemaphoreType.DMA,
        ],
        compiler_params=pltpu.CompilerParams(needs_layout_passes=False),
    )(bpk, xflat, cm, zeros)


def _slabify(x):
    S = x.shape[1] // 128
    return jnp.transpose(x.reshape(N, S, 128), (1, 0, 2)).reshape(S * N, 128)


def _unslabify(xs, S):
    return jnp.transpose(xs.reshape(S, N, 128), (1, 0, 2)).reshape(N, S * 128)


def _mlp_body(h_ref, w1_ref, b1_ref, w2_ref, b2_ref, w3_ref, b3_ref, out_ref):
    h = h_ref[...]
    h = jnp.maximum(jnp.dot(h, w1_ref[...], preferred_element_type=jnp.float32) + b1_ref[...], 0.0)
    h = jnp.maximum(jnp.dot(h, w2_ref[...], preferred_element_type=jnp.float32) + b2_ref[...], 0.0)
    logits = jnp.dot(h, w3_ref[...], preferred_element_type=jnp.float32) + b3_ref[...]
    m = jnp.max(logits, axis=-1, keepdims=True)
    sh = logits - m
    lse = jnp.log(jnp.sum(jnp.exp(sh), axis=-1, keepdims=True))
    out_ref[...] = sh - lse


def _mlp_head(h, Wm1, bm1, Wm2, bm2, Wm3, bm3):
    return pl.pallas_call(
        _mlp_body,
        out_shape=jax.ShapeDtypeStruct((G, 2), jnp.float32),
    )(h, Wm1, bm1, Wm2, bm2, Wm3, bm3)


def _bn(x, g, b):
    mu = jnp.mean(x, axis=0)
    var = jnp.var(x, axis=0)
    return (x - mu) * jax.lax.rsqrt(var + 1e-5) * g + b


def kernel(x, edge_index, edge_weight, batch, W_rel1, b_rel1, W_root1, g1, be1, W_rel2, b_rel2, W_root2, g2, be2, W_rel3, b_rel3, W_root3, g3, be3, Wm1, bm1, Wm2, bm2, Wm3, bm3):
    src, dst = edge_index[0], edge_index[1]
    ew_bits = lax.bitcast_convert_type(edge_weight, jnp.int32)
    zeros = jnp.zeros((624, 128), jnp.float32)

    bpk, cm = _sc_partition(src, dst, ew_bits)

    agg1 = _sc_agg(x, bpk, cm, zeros, mode="l1", passes=1, out_slabs=1)
    y1 = agg1 @ W_rel1 + b_rel1 + x @ W_root1
    x1 = jax.nn.relu(jnp.concatenate([_bn(y1, g1, be1), x], axis=1))

    a2 = _sc_agg(_slabify(x1), bpk, cm, zeros, mode="slab", passes=1, out_slabs=2)
    agg2 = _unslabify(a2, 2)
    y2 = agg2 @ W_rel2 + b_rel2 + x1 @ W_root2
    x2 = jax.nn.relu(jnp.concatenate([_bn(y2, g2, be2), x1], axis=1))

    a3 = _sc_agg(_slabify(x2), bpk, cm, zeros, mode="slab", passes=2, out_slabs=4)
    agg3 = _unslabify(a3, 4)
    y3 = agg3 @ W_rel3 + b_rel3 + x2 @ W_root3
    x3 = jax.nn.relu(jnp.concatenate([_bn(y3, g3, be3), x2], axis=1))

    x_add = jax.ops.segment_sum(x3, batch, num_segments=G)
    cnt = jax.ops.segment_sum(jnp.ones((x3.shape[0],), x3.dtype), batch, num_segments=G)
    x_mean = x_add / jnp.maximum(cnt, 1.0)[:, None]
    x_max = jax.ops.segment_max(x3, batch, num_segments=G)
    x_max = jnp.where(jnp.isfinite(x_max), x_max, 0.0)
    h = jnp.concatenate([x_add, x_max, x_mean], axis=1)
    return _mlp_head(h, Wm1, bm1, Wm2, bm2, Wm3, bm3)


# speedup vs baseline: 1.0586x; 1.0586x over previous
import functools

import jax
import jax.numpy as jnp
from jax import lax
from jax.experimental import pallas as pl
from jax.experimental.pallas import tpu as pltpu
from jax.experimental.pallas import tpu_sc as plsc

N = 10000
E = 320000
G = 64
NC = 2
NS = 16
NW = NC * NS
CH = 80
CAP = E + 80
EPW = E // NW
HALF = N // 2


def _iota16():
    return lax.broadcasted_iota(jnp.int32, (16,), 0)


def _bucket_of(d16):
    return lax.shift_right_logical((d16 + d16) * 6711, 22)


def _div80(c):
    return lax.shift_right_logical(lax.shift_right_logical(c, 4) * 26215, 17)


def _wid():
    return lax.axis_index("c") * NS + lax.axis_index("s")


def _count_body(dst_hbm, cm_out, dv, stg):
    w = _wid()
    iota = _iota16()

    def chunk(ci, carry):
        lo, hi = carry
        pltpu.sync_copy(dst_hbm.at[pl.ds(w * EPW + ci * 2000, 2000)], dv)

        def vstep(k, c2):
            lo2, hi2 = c2
            b16 = _bucket_of(dv[pl.ds(k * 16, 16)])
            for i in range(16):
                oh = (iota == i).astype(jnp.int32)
                lo2 = lo2 + plsc.all_reduce_population_count(b16 == i) * oh
                hi2 = hi2 + plsc.all_reduce_population_count(b16 == (16 + i)) * oh
            return lo2, hi2

        return lax.fori_loop(0, 125, vstep, (lo, hi))

    z = jnp.zeros((16,), jnp.int32)
    lo, hi = lax.fori_loop(0, EPW // 2000, chunk, (z, z))
    stg[pl.ds(0, 16)] = lo
    stg[pl.ds(16, 16)] = hi
    pltpu.sync_copy(stg, cm_out.at[pl.ds(w * 32, 32)])


def _part_body(src_hbm, dst_hbm, ew_hbm, cm_hbm, bsrc, bdst, bew,
               cmv, sv, dv, wv, slot_v):
    w = _wid()
    iota = _iota16()
    pltpu.sync_copy(cm_hbm, cmv)
    o_lo = iota * CAP
    o_hi = (iota + 16) * CAP

    def accw(wp, carry):
        lo, hi = carry
        m = jnp.where(wp < w, 1, 0)
        lo = lo + cmv[pl.ds(wp * 32, 16)] * m
        hi = hi + cmv[pl.ds(wp * 32 + 16, 16)] * m
        return lo, hi

    o_lo, o_hi = lax.fori_loop(0, NW, accw, (o_lo, o_hi))

    def chunk(ci, carry):
        o_lo, o_hi = carry
        e0 = w * EPW + ci * CH
        pltpu.sync_copy(src_hbm.at[pl.ds(e0, CH)], sv)
        pltpu.sync_copy(dst_hbm.at[pl.ds(e0, CH)], dv)
        pltpu.sync_copy(ew_hbm.at[pl.ds(e0, CH)], wv)
        for k in range(CH // 16):
            sl16 = pl.ds(k * 16, 16)
            b16 = _bucket_of(dv[sl16])
            rank = jnp.zeros((16,), jnp.int32)
            for m in range(1, 16):
                sh = b16.at[jnp.maximum(iota - m, 0)].get(mode="promise_in_bounds")
                rank = rank + jnp.where((iota >= m) & (sh == b16), 1, 0)
            blo = jnp.minimum(b16, 15)
            bhi = jnp.maximum(b16 - 16, 0)
            g_lo = o_lo.at[blo].get(mode="promise_in_bounds")
            g_hi = o_hi.at[bhi].get(mode="promise_in_bounds")
            slot_v[sl16] = jnp.where(b16 < 16, g_lo, g_hi) + rank
            for i in range(16):
                oh = (iota == i).astype(jnp.int32)
                o_lo = o_lo + plsc.all_reduce_population_count(b16 == i) * oh
                o_hi = o_hi + plsc.all_reduce_population_count(b16 == (16 + i)) * oh
        pltpu.sync_copy(sv, bsrc.at[slot_v])
        pltpu.sync_copy(dv, bdst.at[slot_v])
        pltpu.sync_copy(wv, bew.at[slot_v])
        return o_lo, o_hi

    lax.fori_loop(0, EPW // CH, chunk, (o_lo, o_hi))


def _agg_body(bsrc, bdst, bew, x_hbm, cm_hbm, zeros_hbm, out_hbm,
              cmv, idx_v, dl_v, ew_v, grows, srows, acc, sem, *, mode, passes):
    c = lax.axis_index("c")
    s = lax.axis_index("s")
    iota = _iota16()
    pltpu.sync_copy(cm_hbm, cmv.at[pl.ds(0, NW * 32)])

    if mode == "l1":
        zrows, zextra, zoff = 312, 16, 4992
        worows, woextra, wooff = 312, 8, 4992
        trash = HALF
    else:
        zrows, zextra, zoff = 624, 24, 9984
        worows, woextra, wooff = 624, 16, 9984
        trash = N

    for p in range(passes):
        if mode == "l1":
            buckets = [c * 16 + s]
            rowbase = c * HALF
            go = c * 0
            outbase = c * HALF
        else:
            buckets = [2 * s, 2 * s + 1]
            rowbase = c * 0
            sl = NC * p + c
            go = sl * N
            outbase = sl * N

        pltpu.sync_copy(zeros_hbm.at[pl.ds(0, zrows), :],
                        acc.at[pl.ds(s * zrows, zrows), :])

        @pl.when(s == NS - 1)
        def _():
            pltpu.sync_copy(zeros_hbm.at[pl.ds(0, zextra), :],
                            acc.at[pl.ds(zoff, zextra), :])

        plsc.subcore_barrier()

        go_vec = lax.broadcast_in_dim(go, (16,), ())
        rb_vec = lax.broadcast_in_dim(rowbase, (16,), ())

        for b in buckets:
            cnt = jnp.int32(0)
            for wp in range(NW):
                cnt = cnt + cmv[pl.ds(wp * 32 + b, 16)][0]
            nfull = _div80(cnt)
            r = cnt - nfull * 80
            ebase = b * CAP

            def do_chunk(e0, rmask):
                pltpu.sync_copy(bsrc.at[pl.ds(e0, CH)], idx_v)
                pltpu.sync_copy(bdst.at[pl.ds(e0, CH)], dl_v)
                pltpu.sync_copy(bew.at[pl.ds(e0, CH)], ew_v)
                for k in range(CH // 16):
                    sl16 = pl.ds(k * 16, 16)
                    if rmask is None:
                        idx_v[sl16] = idx_v[sl16] + go_vec
                        dl_v[sl16] = dl_v[sl16] - rb_vec
                    else:
                        m = (iota + k * 16) < rmask
                        idx_v[sl16] = jnp.where(m, idx_v[sl16] + go_vec, 0)
                        dl_v[sl16] = jnp.where(m, dl_v[sl16] - rb_vec, trash)
                        ew_v[sl16] = jnp.where(m, ew_v[sl16], 0.0)
                pltpu.async_copy(x_hbm.at[idx_v], grows, sem).wait()

                def egroup(q, carry):
                    evs = ew_v[pl.ds(q * 16, 16)]
                    for rr in range(16):
                        j = q * 16 + rr
                        ev = evs[rr]
                        for f in range(8):
                            fl = pl.ds(f * 16, 16)
                            srows[j, fl] = grows[j, fl] * ev
                    return carry

                lax.fori_loop(0, CH // 16, egroup, 0)
                pltpu.sync_copy(srows, acc.at[dl_v], add=True)

            def full_chunk(ci, carry):
                do_chunk(ebase + ci * CH, None)
                return carry

            lax.fori_loop(0, nfull, full_chunk, 0)
            rv = lax.broadcast_in_dim(r, (16,), ())
            do_chunk(ebase + nfull * CH, rv)

        plsc.subcore_barrier()

        pltpu.sync_copy(acc.at[pl.ds(s * worows, worows), :],
                        out_hbm.at[pl.ds(outbase + s * worows, worows), :])

        @pl.when(s == NS - 1)
        def _():
            pltpu.sync_copy(acc.at[pl.ds(wooff, woextra), :],
                            out_hbm.at[pl.ds(outbase + wooff, woextra), :])
        if p + 1 < passes:
            plsc.subcore_barrier()


_MESH = dict(core_axis_name="c", subcore_axis_name="s",
             num_cores=NC, num_subcores=NS)


def _sc_partition(src, dst, ew):
    cm = pl.kernel(
        _count_body,
        out_type=jax.ShapeDtypeStruct((NW * 32,), jnp.int32),
        mesh=plsc.VectorSubcoreMesh(**_MESH),
        scratch_types=[
            pltpu.VMEM((2000,), jnp.int32),
            pltpu.VMEM((32,), jnp.int32),
        ],
        compiler_params=pltpu.CompilerParams(needs_layout_passes=False),
    )(dst)
    bsrc, bdst, bew = pl.kernel(
        _part_body,
        out_type=(jax.ShapeDtypeStruct((NW * CAP,), jnp.int32),
                  jax.ShapeDtypeStruct((NW * CAP,), jnp.int32),
                  jax.ShapeDtypeStruct((NW * CAP,), jnp.float32)),
        mesh=plsc.VectorSubcoreMesh(**_MESH),
        scratch_types=[
            pltpu.VMEM((NW * 32,), jnp.int32),
            pltpu.VMEM((CH,), jnp.int32),
            pltpu.VMEM((CH,), jnp.int32),
            pltpu.VMEM((CH,), jnp.float32),
            pltpu.VMEM((CH,), jnp.int32),
        ],
        compiler_params=pltpu.CompilerParams(needs_layout_passes=False),
    )(src, dst, ew, cm)
    return bsrc, bdst, bew, cm


def _sc_agg(xflat, bsrc, bdst, bew, cm, zeros, *, mode, passes, out_slabs):
    accr = (HALF + 8) if mode == "l1" else (N + 8)
    body = functools.partial(_agg_body, mode=mode, passes=passes)
    return pl.kernel(
        body,
        out_type=jax.ShapeDtypeStruct((out_slabs * N, 128), jnp.float32),
        mesh=plsc.VectorSubcoreMesh(**_MESH),
        scratch_types=[
            pltpu.VMEM((NW * 32 + 16,), jnp.int32),
            pltpu.VMEM((CH,), jnp.int32),
            pltpu.VMEM((CH,), jnp.int32),
            pltpu.VMEM((CH,), jnp.float32),
            pltpu.VMEM((CH, 128), jnp.float32),
            pltpu.VMEM((CH, 128), jnp.float32),
            pltpu.VMEM_SHARED((accr, 128), jnp.float32),
            pltpu.SemaphoreType.DMA,
        ],
        compiler_params=pltpu.CompilerParams(needs_layout_passes=False),
    )(bsrc, bdst, bew, xflat, cm, zeros)


def _slabify(x):
    S = x.shape[1] // 128
    return jnp.transpose(x.reshape(N, S, 128), (1, 0, 2)).reshape(S * N, 128)


def _unslabify(xs, S):
    return jnp.transpose(xs.reshape(S, N, 128), (1, 0, 2)).reshape(N, S * 128)


def _mlp_body(h_ref, w1_ref, b1_ref, w2_ref, b2_ref, w3_ref, b3_ref, out_ref):
    h = h_ref[...]
    h = jnp.maximum(jnp.dot(h, w1_ref[...], preferred_element_type=jnp.float32) + b1_ref[...], 0.0)
    h = jnp.maximum(jnp.dot(h, w2_ref[...], preferred_element_type=jnp.float32) + b2_ref[...], 0.0)
    logits = jnp.dot(h, w3_ref[...], preferred_element_type=jnp.float32) + b3_ref[...]
    m = jnp.max(logits, axis=-1, keepdims=True)
    sh = logits - m
    lse = jnp.log(jnp.sum(jnp.exp(sh), axis=-1, keepdims=True))
    out_ref[...] = sh - lse


def _mlp_head(h, Wm1, bm1, Wm2, bm2, Wm3, bm3):
    return pl.pallas_call(
        _mlp_body,
        out_shape=jax.ShapeDtypeStruct((G, 2), jnp.float32),
    )(h, Wm1, bm1, Wm2, bm2, Wm3, bm3)


def _bn(x, g, b):
    mu = jnp.mean(x, axis=0)
    var = jnp.var(x, axis=0)
    return (x - mu) * jax.lax.rsqrt(var + 1e-5) * g + b


def kernel(x, edge_index, edge_weight, batch, W_rel1, b_rel1, W_root1, g1, be1, W_rel2, b_rel2, W_root2, g2, be2, W_rel3, b_rel3, W_root3, g3, be3, Wm1, bm1, Wm2, bm2, Wm3, bm3):
    src, dst = edge_index[0], edge_index[1]
    ew = edge_weight
    zeros = jnp.zeros((624, 128), jnp.float32)

    bsrc, bdst, bew, cm = _sc_partition(src, dst, ew)

    agg1 = _sc_agg(x, bsrc, bdst, bew, cm, zeros, mode="l1", passes=1, out_slabs=1)
    y1 = agg1 @ W_rel1 + b_rel1 + x @ W_root1
    x1 = jax.nn.relu(jnp.concatenate([_bn(y1, g1, be1), x], axis=1))

    a2 = _sc_agg(_slabify(x1), bsrc, bdst, bew, cm, zeros, mode="slab", passes=1, out_slabs=2)
    agg2 = _unslabify(a2, 2)
    y2 = agg2 @ W_rel2 + b_rel2 + x1 @ W_root2
    x2 = jax.nn.relu(jnp.concatenate([_bn(y2, g2, be2), x1], axis=1))

    a3 = _sc_agg(_slabify(x2), bsrc, bdst, bew, cm, zeros, mode="slab", passes=2, out_slabs=4)
    agg3 = _unslabify(a3, 4)
    y3 = agg3 @ W_rel3 + b_rel3 + x2 @ W_root3
    x3 = jax.nn.relu(jnp.concatenate([_bn(y3, g3, be3), x2], axis=1))

    x_add = jax.ops.segment_sum(x3, batch, num_segments=G)
    cnt = jax.ops.segment_sum(jnp.ones((x3.shape[0],), x3.dtype), batch, num_segments=G)
    x_mean = x_add / jnp.maximum(cnt, 1.0)[:, None]
    x_max = jax.ops.segment_max(x3, batch, num_segments=G)
    x_max = jnp.where(jnp.isfinite(x_max), x_max, 0.0)
    h = jnp.concatenate([x_add, x_max, x_mean], axis=1)
    return _mlp_head(h, Wm1, bm1, Wm2, bm2, Wm3, bm3)
